# baseline (speedup 1.0000x reference)
import functools

import jax
import jax.numpy as jnp
from jax import lax
from jax.experimental import pallas as pl
from jax.experimental.pallas import tpu as pltpu
from jax.experimental.pallas import tpu_sc as plsc

NC = 1
NS = 16
NW = NC * NS
K = 128
R0, R1 = 4, 1

F32 = jnp.float32


def _pad_nodes(n):
    return ((n + 1 + 2047) // 2048) * 2048


NBUF = 8


def _chunk_consts(NCH):
    if NC == 1:
        q0, q1 = NCH // NS, 0
    else:
        q0 = NCH * R0 // ((R0 + R1) * NS)
        q1 = (NCH - NS * q0) // NS
    rem = NCH - NS * (q0 + q1)
    assert 0 <= rem <= NS
    CWmax = q0 + 1
    return q0, q1, rem, CWmax


def _chunk_range(c, s, NCH):
    q0, q1, rem, CWmax = _chunk_consts(NCH)
    start0 = s * q0 + jnp.minimum(s, rem)
    cnt0 = q0 + jnp.where(s < rem, 1, 0)
    tot0 = NS * q0 + rem
    start1 = tot0 + s * q1
    cnt1 = jnp.full((), q1, jnp.int32)
    start = jnp.where(c == 0, start0, start1)
    cnt = jnp.where(c == 0, cnt0, cnt1)
    base = jnp.minimum(start, NCH - CWmax)
    off = start - base
    return CWmax, base, off, cnt


def _make_deg_kernel(Np, NCH):
    mesh = plsc.VectorSubcoreMesh(core_axis_name="c", subcore_axis_name="s",
                                  num_cores=NC)
    rows_per = Np // NS
    CWmax = _chunk_consts(NCH)[3]

    @functools.partial(
        pl.kernel,
        mesh=mesh,
        out_type=jax.ShapeDtypeStruct((NC * Np,), F32),
        compiler_params=pltpu.CompilerParams(use_tc_tiling_on_sc=False),
        scratch_types=[
            pltpu.VMEM((CWmax, K), jnp.int32),
            pltpu.VMEM((K,), F32),
            pltpu.VMEM_SHARED((Np,), F32),
            pltpu.SemaphoreType.DMA,
        ],
    )
    def deg_kernel(edges_hbm, zeros_hbm, out_hbm, dst_idx, ones, acc, sem):
        c = lax.axis_index("c")
        s = lax.axis_index("s")
        _, base, off, cnt = _chunk_range(c, s, NCH)
        pltpu.sync_copy(edges_hbm.at[1].at[pl.ds(base, CWmax)], dst_idx)
        for i in range(K // 16):
            ones[pl.ds(i * 16, 16)] = jnp.ones((16,), F32)
        pltpu.sync_copy(zeros_hbm.at[pl.ds(s * rows_per, rows_per)],
                        acc.at[pl.ds(s * rows_per, rows_per)])
        plsc.subcore_barrier()

        def step(it, carry):
            for b in range(NBUF):
                j = it * NBUF + b

                @pl.when(j < cnt)
                def _():
                    pltpu.async_copy(ones, acc.at[dst_idx.at[off + j]],
                                     sem, add=True)
            for b in range(NBUF):
                j = it * NBUF + b

                @pl.when(j < cnt)
                def _():
                    pltpu.make_async_copy(ones, acc.at[dst_idx.at[0]],
                                          sem).wait()
            return carry

        lax.fori_loop(0, -(-CWmax // NBUF), step, 0)
        plsc.subcore_barrier()
        pltpu.sync_copy(acc.at[pl.ds(s * rows_per, rows_per)],
                        out_hbm.at[pl.ds(c * Np + s * rows_per, rows_per)])

    return deg_kernel


def _make_agg_kernel(Np, H, NCH):
    mesh = plsc.VectorSubcoreMesh(core_axis_name="c", subcore_axis_name="s",
                                  num_cores=NC)
    rows_per = Np // NS
    NG = NBUF // 2
    CWmax = _chunk_consts(NCH)[3]

    @functools.partial(
        pl.kernel,
        mesh=mesh,
        out_type=jax.ShapeDtypeStruct((NC, Np, H), F32),
        compiler_params=pltpu.CompilerParams(use_tc_tiling_on_sc=False),
        scratch_types=(
            [pltpu.VMEM((CWmax, K), jnp.int32)] * 2
            + [pltpu.VMEM((K, H), F32)] * NBUF
            + [pltpu.VMEM_SHARED((Np, H), F32)]
            + [pltpu.SemaphoreType.DMA] * (2 * NBUF)
        ),
    )
    def agg_kernel(g_hbm, edges_hbm, zeros_hbm, out_hbm,
                   src_idx, dst_idx, *rest):
        bufs = rest[:NBUF]
        acc = rest[NBUF]
        gsem = rest[NBUF + 1:NBUF + 1 + NBUF]
        ssem = rest[NBUF + 1 + NBUF:]
        c = lax.axis_index("c")
        s = lax.axis_index("s")
        _, base, off, cnt = _chunk_range(c, s, NCH)
        pltpu.sync_copy(edges_hbm.at[0].at[pl.ds(base, CWmax)], src_idx)
        pltpu.sync_copy(edges_hbm.at[1].at[pl.ds(base, CWmax)], dst_idx)
        pltpu.sync_copy(zeros_hbm.at[pl.ds(s * rows_per, rows_per)],
                        acc.at[pl.ds(s * rows_per, rows_per)])
        plsc.subcore_barrier()

        def gather_start(j, b):
            pltpu.async_copy(g_hbm.at[src_idx.at[off + j]], bufs[b], gsem[b])

        def gather_wait(b):
            pltpu.make_async_copy(g_hbm.at[src_idx.at[0]], bufs[b],
                                  gsem[b]).wait()

        def scatter_start(j, b):
            pltpu.async_copy(bufs[b], acc.at[dst_idx.at[off + j]], ssem[b],
                             add=True)

        def scatter_wait(b):
            pltpu.make_async_copy(bufs[b], acc.at[dst_idx.at[0]],
                                  ssem[b]).wait()

        for b in range(NBUF):
            gather_start(b, b)

        def step(it, carry):
            for half in range(2):
                bb = it * NBUF + half * NG
                for i in range(NG):
                    b = half * NG + i
                    j = bb + i

                    @pl.when(j < cnt)
                    def _():
                        gather_wait(b)
                        scatter_start(j, b)
                for i in range(NG):
                    b = half * NG + i
                    j = bb + i

                    @pl.when(j < cnt)
                    def _():
                        scatter_wait(b)
                    j2 = j + NBUF

                    @pl.when(j2 < cnt)
                    def _():
                        gather_start(j2, b)
            return carry

        lax.fori_loop(0, -(-CWmax // NBUF), step, 0)
        plsc.subcore_barrier()
        pltpu.sync_copy(acc.at[pl.ds(s * rows_per, rows_per)],
                        out_hbm.at[c, pl.ds(s * rows_per, rows_per)])

    return agg_kernel


def _first_layer_body(x_ref, w_ref, degp_ref, g_ref, dinv_ref):
    deg = 1.0
    for i in range(NC):
        deg = deg + degp_ref[i]
    dv = lax.rsqrt(deg)
    dinv_ref[...] = dv
    h = jnp.dot(x_ref[...], w_ref[...], preferred_element_type=F32)
    g_ref[...] = dv * h


def _mid_layer_body(pp_ref, g_ref, dinv_ref, b_ref, w_ref, xl_ref, gn_ref):
    dv = dinv_ref[...]
    agg = g_ref[...]
    for i in range(NC):
        agg = agg + pp_ref[i]
    xl = jnp.maximum(dv * agg + b_ref[...], 0.0)
    xl_ref[...] = xl
    gn_ref[...] = dv * jnp.dot(xl, w_ref[...], preferred_element_type=F32)


def _final_body(pp_ref, g_ref, dinv_ref, b_ref, x1_ref, x2_ref,
                wl1_ref, wl2_ref, wl3_ref, blin_ref, out_ref):
    dv = dinv_ref[...]
    agg = g_ref[...]
    for i in range(NC):
        agg = agg + pp_ref[i]
    x3 = dv * agg + b_ref[...]
    logits = (jnp.dot(x1_ref[...], wl1_ref[...], preferred_element_type=F32)
              + jnp.dot(x2_ref[...], wl2_ref[...], preferred_element_type=F32)
              + jnp.dot(x3, wl3_ref[...], preferred_element_type=F32)
              + blin_ref[...])
    m = jnp.max(logits, axis=1, keepdims=True)
    z = logits - m
    lse = jnp.log(jnp.sum(jnp.exp(z), axis=1, keepdims=True))
    out_ref[...] = z - lse


def kernel(x, edge_index, W1, b1, W2, b2, W3, b3, Wlin, blin):
    N, D = x.shape
    E = edge_index.shape[1]
    H = W1.shape[1]
    C = Wlin.shape[1]
    HP = ((H + 7) // 8) * 8
    HE = HP - H
    W1p = jnp.pad(W1, ((0, 0), (0, HE)))
    W2p = jnp.pad(W2, ((0, HE), (0, HE)))
    W3p = jnp.pad(W3, ((0, HE), (0, HE)))
    b1p = jnp.pad(b1, (0, HE))
    b2p = jnp.pad(b2, (0, HE))
    b3p = jnp.pad(b3, (0, HE))
    Wl1 = jnp.pad(Wlin[0:H], ((0, HE), (0, 0)))
    Wl2 = jnp.pad(Wlin[H:2 * H], ((0, HE), (0, 0)))
    Wl3 = jnp.pad(Wlin[2 * H:3 * H], ((0, HE), (0, 0)))
    Np = _pad_nodes(N)
    assert E % K == 0
    NCH = E // K
    edges3 = edge_index.reshape(2, NCH, K)

    zeros_h = jnp.zeros((Np, HP), F32)
    zeros_1 = jnp.zeros((Np,), F32)

    deg_k = _make_deg_kernel(Np, NCH)
    agg_k = _make_agg_kernel(Np, HP, NCH)

    degp = deg_k(edges3, zeros_1)
    degp3 = degp.reshape(NC, Np, 1)

    BR = 10000
    grid = (N // BR,)

    g1, dinv = pl.pallas_call(
        _first_layer_body,
        grid=grid,
        in_specs=[
            pl.BlockSpec((BR, D), lambda i: (i, 0)),
            pl.BlockSpec((D, HP), lambda i: (0, 0)),
            pl.BlockSpec((NC, BR, 1), lambda i: (0, i, 0)),
        ],
        out_specs=[
            pl.BlockSpec((BR, HP), lambda i: (i, 0)),
            pl.BlockSpec((BR, 1), lambda i: (i, 0)),
        ],
        out_shape=[
            jax.ShapeDtypeStruct((N, HP), F32),
            jax.ShapeDtypeStruct((N, 1), F32),
        ],
    )(x, W1p, degp3)

    def mid_layer(g, b, w_next):
        part = agg_k(g, edges3, zeros_h)
        return pl.pallas_call(
            _mid_layer_body,
            grid=grid,
            in_specs=[
                pl.BlockSpec((NC, BR, HP), lambda i: (0, i, 0)),
                pl.BlockSpec((BR, HP), lambda i: (i, 0)),
                pl.BlockSpec((BR, 1), lambda i: (i, 0)),
                pl.BlockSpec((1, HP), lambda i: (0, 0)),
                pl.BlockSpec((HP, HP), lambda i: (0, 0)),
            ],
            out_specs=[
                pl.BlockSpec((BR, HP), lambda i: (i, 0)),
                pl.BlockSpec((BR, HP), lambda i: (i, 0)),
            ],
            out_shape=[
                jax.ShapeDtypeStruct((N, HP), F32),
                jax.ShapeDtypeStruct((N, HP), F32),
            ],
        )(part, g, dinv, b.reshape(1, HP), w_next)

    x1, g2 = mid_layer(g1, b1p, W2p)
    x2, g3 = mid_layer(g2, b2p, W3p)

    part3 = agg_k(g3, edges3, zeros_h)
    out = pl.pallas_call(
        _final_body,
        grid=grid,
        in_specs=[
            pl.BlockSpec((NC, BR, HP), lambda i: (0, i, 0)),
            pl.BlockSpec((BR, HP), lambda i: (i, 0)),
            pl.BlockSpec((BR, 1), lambda i: (i, 0)),
            pl.BlockSpec((1, HP), lambda i: (0, 0)),
            pl.BlockSpec((BR, HP), lambda i: (i, 0)),
            pl.BlockSpec((BR, HP), lambda i: (i, 0)),
            pl.BlockSpec((HP, C), lambda i: (0, 0)),
            pl.BlockSpec((HP, C), lambda i: (0, 0)),
            pl.BlockSpec((HP, C), lambda i: (0, 0)),
            pl.BlockSpec((1, C), lambda i: (0, 0)),
        ],
        out_specs=pl.BlockSpec((BR, C), lambda i: (i, 0)),
        out_shape=jax.ShapeDtypeStruct((N, C), F32),
    )(part3, g3, dinv, b3p.reshape(1, HP), x1, x2,
      Wl1, Wl2, Wl3, blin.reshape(1, C))
    return out


# speedup vs baseline: 1.0361x; 1.0361x over previous
import functools

import jax
import jax.numpy as jnp
from jax import lax
from jax.experimental import pallas as pl
from jax.experimental.pallas import tpu as pltpu
from jax.experimental.pallas import tpu_sc as plsc

NC = 1
NS = 16
NW = NC * NS
K = 128
R0, R1 = 4, 1

F32 = jnp.float32


def _pad_nodes(n):
    return ((n + 1 + 2047) // 2048) * 2048


NBUF = 8


def _chunk_consts(NCH):
    if NC == 1:
        q0, q1 = NCH // NS, 0
    else:
        q0 = NCH * R0 // ((R0 + R1) * NS)
        q1 = (NCH - NS * q0) // NS
    rem = NCH - NS * (q0 + q1)
    assert 0 <= rem <= NS
    CWmax = q0 + 1
    return q0, q1, rem, CWmax


def _chunk_range(c, s, NCH):
    q0, q1, rem, CWmax = _chunk_consts(NCH)
    start0 = s * q0 + jnp.minimum(s, rem)
    cnt0 = q0 + jnp.where(s < rem, 1, 0)
    tot0 = NS * q0 + rem
    start1 = tot0 + s * q1
    cnt1 = jnp.full((), q1, jnp.int32)
    start = jnp.where(c == 0, start0, start1)
    cnt = jnp.where(c == 0, cnt0, cnt1)
    base = jnp.minimum(start, NCH - CWmax)
    off = start - base
    return CWmax, base, off, cnt


def _make_deg_kernel(Np, NCH):
    mesh = plsc.VectorSubcoreMesh(core_axis_name="c", subcore_axis_name="s",
                                  num_cores=NC)
    rows_per = Np // NS
    CWmax = _chunk_consts(NCH)[3]

    @functools.partial(
        pl.kernel,
        mesh=mesh,
        out_type=jax.ShapeDtypeStruct((NC * Np,), F32),
        compiler_params=pltpu.CompilerParams(use_tc_tiling_on_sc=False),
        scratch_types=[
            pltpu.VMEM((CWmax, K), jnp.int32),
            pltpu.VMEM((K,), F32),
            pltpu.VMEM_SHARED((Np,), F32),
            pltpu.SemaphoreType.DMA,
        ],
    )
    def deg_kernel(edges_hbm, zeros_hbm, out_hbm, dst_idx, ones, acc, sem):
        c = lax.axis_index("c")
        s = lax.axis_index("s")
        _, base, off, cnt = _chunk_range(c, s, NCH)
        pltpu.sync_copy(edges_hbm.at[1].at[pl.ds(base, CWmax)], dst_idx)
        for i in range(K // 16):
            ones[pl.ds(i * 16, 16)] = jnp.ones((16,), F32)
        pltpu.sync_copy(zeros_hbm.at[pl.ds(s * rows_per, rows_per)],
                        acc.at[pl.ds(s * rows_per, rows_per)])
        plsc.subcore_barrier()

        def step(it, carry):
            for b in range(NBUF):
                j = it * NBUF + b

                @pl.when(j < cnt)
                def _():
                    pltpu.async_copy(ones, acc.at[dst_idx.at[off + j]],
                                     sem, add=True)
            for b in range(NBUF):
                j = it * NBUF + b

                @pl.when(j < cnt)
                def _():
                    pltpu.make_async_copy(ones, acc.at[dst_idx.at[0]],
                                          sem).wait()
            return carry

        lax.fori_loop(0, -(-CWmax // NBUF), step, 0)
        plsc.subcore_barrier()
        pltpu.sync_copy(acc.at[pl.ds(s * rows_per, rows_per)],
                        out_hbm.at[pl.ds(c * Np + s * rows_per, rows_per)])

    return deg_kernel


def _make_agg_kernel(Np, H, NCH):
    mesh = plsc.VectorSubcoreMesh(core_axis_name="c", subcore_axis_name="s",
                                  num_cores=NC)
    rows_per = Np // NS
    NG = NBUF // 2
    CWmax = _chunk_consts(NCH)[3]

    @functools.partial(
        pl.kernel,
        mesh=mesh,
        out_type=jax.ShapeDtypeStruct((NC, Np, H), F32),
        compiler_params=pltpu.CompilerParams(use_tc_tiling_on_sc=False),
        scratch_types=(
            [pltpu.VMEM((CWmax, K), jnp.int32)] * 2
            + [pltpu.VMEM((K, H), F32)] * NBUF
            + [pltpu.VMEM_SHARED((Np, H), F32)]
            + [pltpu.SemaphoreType.DMA] * (2 * NBUF)
        ),
    )
    def agg_kernel(g_hbm, edges_hbm, zeros_hbm, out_hbm,
                   src_idx, dst_idx, *rest):
        bufs = rest[:NBUF]
        acc = rest[NBUF]
        gsem = rest[NBUF + 1:NBUF + 1 + NBUF]
        ssem = rest[NBUF + 1 + NBUF:]
        c = lax.axis_index("c")
        s = lax.axis_index("s")
        _, base, off, cnt = _chunk_range(c, s, NCH)
        pltpu.sync_copy(edges_hbm.at[0].at[pl.ds(base, CWmax)], src_idx)
        pltpu.sync_copy(edges_hbm.at[1].at[pl.ds(base, CWmax)], dst_idx)
        pltpu.sync_copy(zeros_hbm.at[pl.ds(s * rows_per, rows_per)],
                        acc.at[pl.ds(s * rows_per, rows_per)])
        plsc.subcore_barrier()

        def gather_start(j, b):
            pltpu.async_copy(g_hbm.at[src_idx.at[off + j]], bufs[b], gsem[b])

        def gather_wait(b):
            pltpu.make_async_copy(g_hbm.at[src_idx.at[0]], bufs[b],
                                  gsem[b]).wait()

        def scatter_start(j, b):
            pltpu.async_copy(bufs[b], acc.at[dst_idx.at[off + j]], ssem[b],
                             add=True)

        def scatter_wait(b):
            pltpu.make_async_copy(bufs[b], acc.at[dst_idx.at[0]],
                                  ssem[b]).wait()

        for b in range(NBUF):
            gather_start(b, b)

        def step(it, carry):
            for half in range(2):
                bb = it * NBUF + half * NG
                for i in range(NG):
                    b = half * NG + i
                    j = bb + i

                    @pl.when(j < cnt)
                    def _():
                        gather_wait(b)
                        scatter_start(j, b)
                for i in range(NG):
                    b = half * NG + i
                    j = bb + i

                    @pl.when(j < cnt)
                    def _():
                        scatter_wait(b)
                    j2 = j + NBUF

                    @pl.when(j2 < cnt)
                    def _():
                        gather_start(j2, b)
            return carry

        lax.fori_loop(0, -(-CWmax // NBUF), step, 0)
        plsc.subcore_barrier()
        pltpu.sync_copy(acc.at[pl.ds(s * rows_per, rows_per)],
                        out_hbm.at[c, pl.ds(s * rows_per, rows_per)])

    return agg_kernel


def _first_layer_body(x_ref, w_ref, degp_ref, g_ref, dinv_ref):
    deg = 1.0
    for i in range(NC):
        deg = deg + degp_ref[i]
    dv = lax.rsqrt(deg)
    dinv_ref[...] = dv
    h = jnp.dot(x_ref[...], w_ref[...], preferred_element_type=F32)
    g_ref[...] = dv * h


def _mid_layer_body(pp_ref, g_ref, dinv_ref, b_ref, w_ref, xl_ref, gn_ref):
    dv = dinv_ref[...]
    agg = g_ref[...]
    for i in range(NC):
        agg = agg + pp_ref[i]
    xl = jnp.maximum(dv * agg + b_ref[...], 0.0)
    xl_ref[...] = xl
    gn_ref[...] = dv * jnp.dot(xl, w_ref[...], preferred_element_type=F32)


def _final_body(pp_ref, g_ref, dinv_ref, b_ref, x1_ref, x2_ref,
                wl1_ref, wl2_ref, wl3_ref, blin_ref, out_ref):
    dv = dinv_ref[...]
    agg = g_ref[...]
    for i in range(NC):
        agg = agg + pp_ref[i]
    x3 = dv * agg + b_ref[...]
    logits = (jnp.dot(x1_ref[...], wl1_ref[...], preferred_element_type=F32)
              + jnp.dot(x2_ref[...], wl2_ref[...], preferred_element_type=F32)
              + jnp.dot(x3, wl3_ref[...], preferred_element_type=F32)
              + blin_ref[...])
    m = jnp.max(logits, axis=1, keepdims=True)
    z = logits - m
    lse = jnp.log(jnp.sum(jnp.exp(z), axis=1, keepdims=True))
    out_ref[...] = z - lse


def kernel(x, edge_index, W1, b1, W2, b2, W3, b3, Wlin, blin):
    N, D = x.shape
    E = edge_index.shape[1]
    H = W1.shape[1]
    C = Wlin.shape[1]
    HP = ((H + 7) // 8) * 8
    HE = HP - H
    W1p = jnp.pad(W1, ((0, 0), (0, HE)))
    W2p = jnp.pad(W2, ((0, HE), (0, HE)))
    W3p = jnp.pad(W3, ((0, HE), (0, HE)))
    b1p = jnp.pad(b1, (0, HE))
    b2p = jnp.pad(b2, (0, HE))
    b3p = jnp.pad(b3, (0, HE))
    Wl1 = jnp.pad(Wlin[0:H], ((0, HE), (0, 0)))
    Wl2 = jnp.pad(Wlin[H:2 * H], ((0, HE), (0, 0)))
    Wl3 = jnp.pad(Wlin[2 * H:3 * H], ((0, HE), (0, 0)))
    Np = _pad_nodes(N)
    assert E % K == 0
    NCH = E // K
    edges3 = edge_index.reshape(2, NCH, K)

    zeros_h = jnp.zeros((Np, HP), F32)
    zeros_1 = jnp.zeros((Np,), F32)

    deg_k = _make_deg_kernel(Np, NCH)
    agg_k = _make_agg_kernel(Np, HP, NCH)

    degp = deg_k(edges3, zeros_1)
    degp3 = degp.reshape(NC, Np, 1)

    BR = 5000
    grid = (N // BR,)

    g1, dinv = pl.pallas_call(
        _first_layer_body,
        grid=grid,
        in_specs=[
            pl.BlockSpec((BR, D), lambda i: (i, 0)),
            pl.BlockSpec((D, HP), lambda i: (0, 0)),
            pl.BlockSpec((NC, BR, 1), lambda i: (0, i, 0)),
        ],
        out_specs=[
            pl.BlockSpec((BR, HP), lambda i: (i, 0)),
            pl.BlockSpec((BR, 1), lambda i: (i, 0)),
        ],
        out_shape=[
            jax.ShapeDtypeStruct((N, HP), F32),
            jax.ShapeDtypeStruct((N, 1), F32),
        ],
    )(x, W1p, degp3)

    def mid_layer(g, b, w_next):
        part = agg_k(g, edges3, zeros_h)
        return pl.pallas_call(
            _mid_layer_body,
            grid=grid,
            in_specs=[
                pl.BlockSpec((NC, BR, HP), lambda i: (0, i, 0)),
                pl.BlockSpec((BR, HP), lambda i: (i, 0)),
                pl.BlockSpec((BR, 1), lambda i: (i, 0)),
                pl.BlockSpec((1, HP), lambda i: (0, 0)),
                pl.BlockSpec((HP, HP), lambda i: (0, 0)),
            ],
            out_specs=[
                pl.BlockSpec((BR, HP), lambda i: (i, 0)),
                pl.BlockSpec((BR, HP), lambda i: (i, 0)),
            ],
            out_shape=[
                jax.ShapeDtypeStruct((N, HP), F32),
                jax.ShapeDtypeStruct((N, HP), F32),
            ],
        )(part, g, dinv, b.reshape(1, HP), w_next)

    x1, g2 = mid_layer(g1, b1p, W2p)
    x2, g3 = mid_layer(g2, b2p, W3p)

    part3 = agg_k(g3, edges3, zeros_h)
    out = pl.pallas_call(
        _final_body,
        grid=grid,
        in_specs=[
            pl.BlockSpec((NC, BR, HP), lambda i: (0, i, 0)),
            pl.BlockSpec((BR, HP), lambda i: (i, 0)),
            pl.BlockSpec((BR, 1), lambda i: (i, 0)),
            pl.BlockSpec((1, HP), lambda i: (0, 0)),
            pl.BlockSpec((BR, HP), lambda i: (i, 0)),
            pl.BlockSpec((BR, HP), lambda i: (i, 0)),
            pl.BlockSpec((HP, C), lambda i: (0, 0)),
            pl.BlockSpec((HP, C), lambda i: (0, 0)),
            pl.BlockSpec((HP, C), lambda i: (0, 0)),
            pl.BlockSpec((1, C), lambda i: (0, 0)),
        ],
        out_specs=pl.BlockSpec((BR, C), lambda i: (i, 0)),
        out_shape=jax.ShapeDtypeStruct((N, C), F32),
    )(part3, g3, dinv, b3p.reshape(1, HP), x1, x2,
      Wl1, Wl2, Wl3, blin.reshape(1, C))
    return out
